# baseline (speedup 1.0000x reference)
import functools

import jax
import jax.numpy as jnp
from jax import lax
from jax.experimental import pallas as pl
from jax.experimental.pallas import tpu as pltpu
from jax.experimental.pallas import tpu_sc as plsc


def _sc_geometry():
    try:
        info = plsc.get_sparse_core_info()
        return info.num_cores, info.num_subcores
    except Exception:
        return 2, 16


def _proj_body(pre_ref, scale_ref, shift_ref, w_ref, proj_ref, hn_ref):
    r = pl.program_id(0)

    @pl.when(r == 0)
    def _():
        hn_ref[...] = pre_ref[...] * scale_ref[...] + shift_ref[...]

    proj_ref[...] = jnp.dot(hn_ref[...], w_ref[0],
                            preferred_element_type=jnp.float32)


def _make_proj_call(N, D, R):
    return pl.pallas_call(
        _proj_body,
        grid=(R + 1,),
        in_specs=[
            pl.BlockSpec((N, D), lambda r: (0, 0)),
            pl.BlockSpec((1, D), lambda r: (0, 0)),
            pl.BlockSpec((1, D), lambda r: (0, 0)),
            pl.BlockSpec((1, D, D), lambda r: (r, 0, 0)),
        ],
        out_specs=pl.BlockSpec((N, D), lambda r: (r, 0)),
        out_shape=jax.ShapeDtypeStruct(((R + 1) * N, D), jnp.float32),
        scratch_shapes=[pltpu.VMEM((N, D), jnp.float32)],
    )


_CH = 40
_NG = 5


def _make_sc_call(N, D, E):
    NC, NS = _sc_geometry()
    NW = NC * NS
    EPW = E // NW
    CH = _CH
    NCHK = EPW // CH
    NG = _NG
    GRP = NCHK // NG
    NB_R = 6
    assert EPW * NW == E and NCHK * CH == EPW and GRP * NG == NCHK
    ZR = 40
    NCK = N // ZR
    assert NCK * ZR == N
    JMAX = (NCK + NS - 1) // NS

    mesh = plsc.VectorSubcoreMesh(core_axis_name="c", subcore_axis_name="s",
                                  num_cores=NC, num_subcores=NS)

    @functools.partial(
        pl.kernel,
        out_type=jax.ShapeDtypeStruct((NC * NCK, ZR, D), jnp.float32),
        mesh=mesh,
        scratch_types=[
            pltpu.VMEM((GRP, CH), jnp.int32),
            pltpu.VMEM((GRP, CH), jnp.int32),
            pltpu.VMEM((ZR, D), jnp.float32),
            pltpu.VMEM_SHARED((N, D), jnp.float32),
        ] + [pltpu.VMEM((CH, D), jnp.float32) for _ in range(NB_R)]
          + [pltpu.SemaphoreType.DMA for _ in range(2 * NB_R)],
    )
    def sc_fn(proj_hbm, gidx_hbm, dst_hbm, out_hbm,
              gix_v, dix_v, stage_v, acc_sh, *ring):
        rows = ring[:NB_R]
        gsems = ring[NB_R:2 * NB_R]
        ssems = ring[2 * NB_R:3 * NB_R]
        c = lax.axis_index("c")
        s = lax.axis_index("s")
        wid = s * NC + c

        zeros16 = jnp.zeros((16,), jnp.float32)

        def zrow(r, _):
            for k in range(D // 16):
                stage_v[r, pl.ds(k * 16, 16)] = zeros16
            return 0

        lax.fori_loop(0, ZR, zrow, 0)
        for j in range(JMAX):
            ck = j * NS + s

            @pl.when(ck < NCK)
            def _():
                off = pl.multiple_of(ck * ZR, 8)
                pltpu.sync_copy(stage_v, acc_sh.at[pl.ds(off, ZR)])

        plsc.subcore_barrier()

        HLF = NB_R // 2

        def group(g, _):
            pltpu.sync_copy(gidx_hbm.at[wid, g], gix_v)
            pltpu.sync_copy(dst_hbm.at[wid, g], dix_v)
            for k in range(NB_R):
                pltpu.async_copy(proj_hbm.at[gix_v.at[k]], rows[k], gsems[k])

            def step(u, _):
                for k in range(NB_R):
                    t = NB_R * u + k

                    @pl.when(t < GRP)
                    def _():
                        pltpu.make_async_copy(
                            proj_hbm.at[gix_v.at[t]], rows[k],
                            gsems[k]).wait()
                        pltpu.async_copy(rows[k], acc_sh.at[dix_v.at[t]],
                                         ssems[k], add=True)

                    tp = t - HLF
                    kp = (k + HLF) % NB_R

                    @pl.when((0 <= tp) & (tp < GRP))
                    def _():
                        pltpu.make_async_copy(
                            rows[kp], acc_sh.at[dix_v.at[tp]],
                            ssems[kp]).wait()

                    @pl.when((0 <= tp) & (tp + NB_R < GRP))
                    def _():
                        pltpu.async_copy(
                            proj_hbm.at[gix_v.at[tp + NB_R]], rows[kp],
                            gsems[kp])

                return 0

            lax.fori_loop(0, (GRP + HLF + NB_R - 1) // NB_R, step, 0)
            return 0

        lax.fori_loop(0, NG, group, 0)
        plsc.subcore_barrier()

        for j in range(JMAX):
            ck = j * NS + s

            @pl.when(ck < NCK)
            def _():
                off = pl.multiple_of(ck * ZR, 8)
                pltpu.sync_copy(acc_sh.at[pl.ds(off, ZR)], stage_v)
                pltpu.sync_copy(stage_v, out_hbm.at[c * NCK + ck])

    return sc_fn


def _combine_body(slabs_ref, selfp_ref, b_ref, gamma_ref, beta_ref,
                  pre_ref, stats_ref, sum_ref, sq_ref, *, nb, n_rows, relu):
    i = pl.program_id(0)
    outp = (slabs_ref[0] + slabs_ref[1] + selfp_ref[...] + b_ref[...])
    if relu:
        outp = jnp.maximum(outp, 0.0)
    pre_ref[...] = outp

    @pl.when(i == 0)
    def _():
        sum_ref[...] = jnp.zeros_like(sum_ref)
        sq_ref[...] = jnp.zeros_like(sq_ref)

    sum_ref[...] += jnp.sum(outp, axis=0, keepdims=True)
    sq_ref[...] += jnp.sum(outp * outp, axis=0, keepdims=True)

    @pl.when(i == nb - 1)
    def _():
        mean = sum_ref[...] / n_rows
        var = sq_ref[...] / n_rows - mean * mean
        scale = gamma_ref[...] * lax.rsqrt(var + 1e-5)
        shift = beta_ref[...] - mean * scale
        stats_ref[...] = jnp.concatenate([scale, shift], axis=0)


def _make_combine_call(N, D, R, BN, relu):
    nb = N // BN
    body = functools.partial(_combine_body, nb=nb, n_rows=float(N), relu=relu)
    return pl.pallas_call(
        body,
        grid=(nb,),
        in_specs=[
            pl.BlockSpec((2, BN, D), lambda n: (0, n, 0)),
            pl.BlockSpec((BN, D), lambda n: (R * nb + n, 0)),
            pl.BlockSpec((1, D), lambda n: (0, 0)),
            pl.BlockSpec((1, D), lambda n: (0, 0)),
            pl.BlockSpec((1, D), lambda n: (0, 0)),
        ],
        out_specs=[
            pl.BlockSpec((BN, D), lambda n: (n, 0)),
            pl.BlockSpec((2, D), lambda n: (0, 0)),
        ],
        out_shape=[
            jax.ShapeDtypeStruct((N, D), jnp.float32),
            jax.ShapeDtypeStruct((2, D), jnp.float32),
        ],
        scratch_shapes=[
            pltpu.VMEM((1, D), jnp.float32),
            pltpu.VMEM((1, D), jnp.float32),
        ],
    )


def _final_body(pre_ref, stats_ref, out_ref):
    out_ref[...] = jnp.maximum(
        pre_ref[...] * stats_ref[0:1, :] + stats_ref[1:2, :], 0.0)


def _make_final_call(N, D, BN):
    nb = N // BN
    return pl.pallas_call(
        _final_body,
        grid=(nb,),
        in_specs=[
            pl.BlockSpec((BN, D), lambda n: (n, 0)),
            pl.BlockSpec((2, D), lambda n: (0, 0)),
        ],
        out_specs=pl.BlockSpec((BN, D), lambda n: (n, 0)),
        out_shape=jax.ShapeDtypeStruct((N, D), jnp.float32),
    )


def kernel(x, edge_index, edge_type, W, W_loop, b, gamma, beta):
    N, D = x.shape
    L, R, _, _ = W.shape
    E = edge_type.shape[0]
    BN = 1000

    NC, NS = _sc_geometry()
    NW = NC * NS
    EPW = E // NW
    CH = _CH
    NCHK = EPW // CH
    NG = _NG

    src = edge_index[0]
    dst = edge_index[1]
    gidx4 = (edge_type * N + src).reshape(NW, NG, NCHK // NG, CH)
    dst4 = dst.reshape(NW, NG, NCHK // NG, CH)

    proj_call = _make_proj_call(N, D, R)
    sc_call = _make_sc_call(N, D, E)
    comb_calls = [_make_combine_call(N, D, R, BN, relu=(l < L - 1))
                  for l in range(L)]
    final_call = _make_final_call(N, D, BN)

    scale = jnp.ones((1, D), jnp.float32)
    shift = jnp.zeros((1, D), jnp.float32)
    pre = x
    for l in range(L):
        w_all = jnp.concatenate([W[l], W_loop[l:l + 1]], axis=0)
        proj_full = proj_call(pre, scale, shift, w_all)
        slabs = sc_call(proj_full, gidx4, dst4).reshape(NC, N, D)
        pre, stats = comb_calls[l](
            slabs, proj_full, b[l:l + 1], gamma[l:l + 1], beta[l:l + 1])
        scale = stats[0:1]
        shift = stats[1:2]
    return final_call(pre, stats)


# speedup vs baseline: 1.1068x; 1.1068x over previous
import functools

import jax
import jax.numpy as jnp
from jax import lax
from jax.experimental import pallas as pl
from jax.experimental.pallas import tpu as pltpu
from jax.experimental.pallas import tpu_sc as plsc


def _sc_geometry():
    try:
        info = plsc.get_sparse_core_info()
        return info.num_cores, info.num_subcores
    except Exception:
        return 2, 16


def _proj_body(pre_ref, scale_ref, shift_ref, w_ref, proj_ref, hn_ref):
    r = pl.program_id(0)

    @pl.when(r == 0)
    def _():
        hn_ref[...] = pre_ref[...] * scale_ref[...] + shift_ref[...]

    proj_ref[...] = jnp.dot(hn_ref[...], w_ref[0],
                            preferred_element_type=jnp.float32)


def _make_proj_call(N, D, R):
    return pl.pallas_call(
        _proj_body,
        grid=(R + 1,),
        in_specs=[
            pl.BlockSpec((N, D), lambda r: (0, 0)),
            pl.BlockSpec((1, D), lambda r: (0, 0)),
            pl.BlockSpec((1, D), lambda r: (0, 0)),
            pl.BlockSpec((1, D, D), lambda r: (r, 0, 0)),
        ],
        out_specs=pl.BlockSpec((N, D), lambda r: (r, 0)),
        out_shape=jax.ShapeDtypeStruct(((R + 1) * N, D), jnp.float32),
        scratch_shapes=[pltpu.VMEM((N, D), jnp.float32)],
    )


_CH = 80
_NG = 5


def _make_sc_call(N, D, E):
    NC, NS = _sc_geometry()
    NW = NC * NS
    EPW = E // NW
    CH = _CH
    NCHK = EPW // CH
    NG = _NG
    GRP = NCHK // NG
    NB_R = 3
    assert EPW * NW == E and NCHK * CH == EPW and GRP * NG == NCHK
    ZR = 40
    NCK = N // ZR
    assert NCK * ZR == N
    JMAX = (NCK + NS - 1) // NS

    mesh = plsc.VectorSubcoreMesh(core_axis_name="c", subcore_axis_name="s",
                                  num_cores=NC, num_subcores=NS)

    @functools.partial(
        pl.kernel,
        out_type=jax.ShapeDtypeStruct((NC * NCK, ZR, D), jnp.float32),
        mesh=mesh,
        scratch_types=[
            pltpu.VMEM((GRP, CH), jnp.int32),
            pltpu.VMEM((GRP, CH), jnp.int32),
            pltpu.VMEM_SHARED((N, D), jnp.float32),
        ] + [pltpu.VMEM((CH, D), jnp.float32) for _ in range(NB_R)]
          + [pltpu.SemaphoreType.DMA for _ in range(2 * NB_R)],
    )
    def sc_fn(proj_hbm, gidx_hbm, dst_hbm, out_hbm,
              gix_v, dix_v, acc_sh, *ring):
        rows = ring[:NB_R]
        gsems = ring[NB_R:2 * NB_R]
        ssems = ring[2 * NB_R:3 * NB_R]
        stage_v = rows[0].at[pl.ds(0, ZR)]
        c = lax.axis_index("c")
        s = lax.axis_index("s")
        wid = s * NC + c

        zeros16 = jnp.zeros((16,), jnp.float32)

        def zrow(r, _):
            for k in range(D // 16):
                rows[0][r, pl.ds(k * 16, 16)] = zeros16
            return 0

        lax.fori_loop(0, ZR, zrow, 0)
        for j in range(JMAX):
            ck = j * NS + s

            @pl.when(ck < NCK)
            def _():
                off = pl.multiple_of(ck * ZR, 8)
                pltpu.sync_copy(stage_v, acc_sh.at[pl.ds(off, ZR)])

        plsc.subcore_barrier()

        def group(g, _):
            pltpu.sync_copy(gidx_hbm.at[wid, g], gix_v)
            pltpu.sync_copy(dst_hbm.at[wid, g], dix_v)
            for k in range(NB_R):
                pltpu.async_copy(proj_hbm.at[gix_v.at[k]], rows[k], gsems[k])

            def step(u, _):
                for k in range(NB_R):
                    t = NB_R * u + k

                    @pl.when(t < GRP)
                    def _():
                        pltpu.make_async_copy(
                            proj_hbm.at[gix_v.at[t]], rows[k],
                            gsems[k]).wait()
                        pltpu.sync_copy(rows[k], acc_sh.at[dix_v.at[t]],
                                        add=True)

                    @pl.when(t + NB_R < GRP)
                    def _():
                        pltpu.async_copy(
                            proj_hbm.at[gix_v.at[t + NB_R]], rows[k],
                            gsems[k])

                return 0

            lax.fori_loop(0, (GRP + NB_R - 1) // NB_R, step, 0)
            return 0

        lax.fori_loop(0, NG, group, 0)
        plsc.subcore_barrier()

        for j in range(JMAX):
            ck = j * NS + s

            @pl.when(ck < NCK)
            def _():
                off = pl.multiple_of(ck * ZR, 8)
                pltpu.sync_copy(acc_sh.at[pl.ds(off, ZR)], stage_v)
                pltpu.sync_copy(stage_v, out_hbm.at[c * NCK + ck])

    return sc_fn


def _combine_body(slabs_ref, selfp_ref, b_ref, gamma_ref, beta_ref,
                  pre_ref, stats_ref, sum_ref, sq_ref, *, nb, n_rows, relu):
    i = pl.program_id(0)
    outp = (slabs_ref[0] + slabs_ref[1] + selfp_ref[...] + b_ref[...])
    if relu:
        outp = jnp.maximum(outp, 0.0)
    pre_ref[...] = outp

    @pl.when(i == 0)
    def _():
        sum_ref[...] = jnp.zeros_like(sum_ref)
        sq_ref[...] = jnp.zeros_like(sq_ref)

    sum_ref[...] += jnp.sum(outp, axis=0, keepdims=True)
    sq_ref[...] += jnp.sum(outp * outp, axis=0, keepdims=True)

    @pl.when(i == nb - 1)
    def _():
        mean = sum_ref[...] / n_rows
        var = sq_ref[...] / n_rows - mean * mean
        scale = gamma_ref[...] * lax.rsqrt(var + 1e-5)
        shift = beta_ref[...] - mean * scale
        stats_ref[...] = jnp.concatenate([scale, shift], axis=0)


def _make_combine_call(N, D, R, BN, relu):
    nb = N // BN
    body = functools.partial(_combine_body, nb=nb, n_rows=float(N), relu=relu)
    return pl.pallas_call(
        body,
        grid=(nb,),
        in_specs=[
            pl.BlockSpec((2, BN, D), lambda n: (0, n, 0)),
            pl.BlockSpec((BN, D), lambda n: (R * nb + n, 0)),
            pl.BlockSpec((1, D), lambda n: (0, 0)),
            pl.BlockSpec((1, D), lambda n: (0, 0)),
            pl.BlockSpec((1, D), lambda n: (0, 0)),
        ],
        out_specs=[
            pl.BlockSpec((BN, D), lambda n: (n, 0)),
            pl.BlockSpec((2, D), lambda n: (0, 0)),
        ],
        out_shape=[
            jax.ShapeDtypeStruct((N, D), jnp.float32),
            jax.ShapeDtypeStruct((2, D), jnp.float32),
        ],
        scratch_shapes=[
            pltpu.VMEM((1, D), jnp.float32),
            pltpu.VMEM((1, D), jnp.float32),
        ],
    )


def _final_body(pre_ref, stats_ref, out_ref):
    out_ref[...] = jnp.maximum(
        pre_ref[...] * stats_ref[0:1, :] + stats_ref[1:2, :], 0.0)


def _make_final_call(N, D, BN):
    nb = N // BN
    return pl.pallas_call(
        _final_body,
        grid=(nb,),
        in_specs=[
            pl.BlockSpec((BN, D), lambda n: (n, 0)),
            pl.BlockSpec((2, D), lambda n: (0, 0)),
        ],
        out_specs=pl.BlockSpec((BN, D), lambda n: (n, 0)),
        out_shape=jax.ShapeDtypeStruct((N, D), jnp.float32),
    )


def kernel(x, edge_index, edge_type, W, W_loop, b, gamma, beta):
    N, D = x.shape
    L, R, _, _ = W.shape
    E = edge_type.shape[0]
    BN = 1000

    NC, NS = _sc_geometry()
    NW = NC * NS
    EPW = E // NW
    CH = _CH
    NCHK = EPW // CH
    NG = _NG

    src = edge_index[0]
    dst = edge_index[1]
    gidx4 = (edge_type * N + src).reshape(NW, NG, NCHK // NG, CH)
    dst4 = dst.reshape(NW, NG, NCHK // NG, CH)

    proj_call = _make_proj_call(N, D, R)
    sc_call = _make_sc_call(N, D, E)
    comb_calls = [_make_combine_call(N, D, R, BN, relu=(l < L - 1))
                  for l in range(L)]
    final_call = _make_final_call(N, D, BN)

    scale = jnp.ones((1, D), jnp.float32)
    shift = jnp.zeros((1, D), jnp.float32)
    pre = x
    for l in range(L):
        w_all = jnp.concatenate([W[l], W_loop[l:l + 1]], axis=0)
        proj_full = proj_call(pre, scale, shift, w_all)
        slabs = sc_call(proj_full, gidx4, dst4).reshape(NC, N, D)
        pre, stats = comb_calls[l](
            slabs, proj_full, b[l:l + 1], gamma[l:l + 1], beta[l:l + 1])
        scale = stats[0:1]
        shift = stats[1:2]
    return final_call(pre, stats)
